# baseline (speedup 1.0000x reference)
import functools

import jax
import jax.numpy as jnp
from jax import lax
from jax.experimental import pallas as pl
from jax.experimental.pallas import tpu as pltpu
from jax.experimental.pallas import tpu_sc as plsc

N = 10000
NPAD = 10240
E = 320000
NC = 2
NS = 16
NW = NC * NS
EPW = E // NW
K = 128
FULL = EPW // K
TAIL = EPW - FULL * K
RPT = NPAD // NS
ZROWS = 64
B = 1000
EB = 2000
NEB = E // EB
EPS = 1e-5


@functools.lru_cache(maxsize=None)
def _make_agg(D):
    mesh = plsc.VectorSubcoreMesh(
        core_axis_name="c", subcore_axis_name="s",
        num_cores=NC, num_subcores=NS)

    @functools.partial(
        pl.kernel,
        out_type=jax.ShapeDtypeStruct((NC, NPAD, D), jnp.float32),
        mesh=mesh,
        compiler_params=pltpu.CompilerParams(use_tc_tiling_on_sc=False),
        scratch_types=[
            pltpu.VMEM((2, K), jnp.int32),
            pltpu.VMEM((2, K), jnp.int32),
            pltpu.VMEM((K, D), jnp.float32),
            pltpu.VMEM((K, D), jnp.float32),
            pltpu.VMEM((TAIL,), jnp.int32),
            pltpu.VMEM((TAIL,), jnp.int32),
            pltpu.VMEM((TAIL, D), jnp.float32),
            pltpu.VMEM((ZROWS, D), jnp.float32),
            pltpu.VMEM_SHARED((NPAD, D), jnp.float32),
            pltpu.SemaphoreType.DMA,
            pltpu.SemaphoreType.DMA,
            pltpu.SemaphoreType.DMA,
            pltpu.SemaphoreType.DMA,
        ],
    )
    def agg(table_hbm, edge_hbm, out_hbm,
            src_v, dst_v, rows_a, rows_b, st_v, dt_v, rows_t, zb_v, acc,
            sia, sib, sga, sgb):
        c = lax.axis_index("c")
        s = lax.axis_index("s")
        wid = c * NS + s

        zeros16 = jnp.zeros((16,), jnp.float32)

        def zrow(j, _):
            def zlane(k, _):
                zb_v[j, pl.ds(k * 16, 16)] = zeros16
                return None
            return lax.fori_loop(0, D // 16, zlane, None)
        lax.fori_loop(0, ZROWS, zrow, None)

        def zcopy(j, _):
            pltpu.sync_copy(zb_v, acc.at[pl.ds(s * RPT + j * ZROWS, ZROWS)])
            return None
        lax.fori_loop(0, RPT // ZROWS, zcopy, None)
        plsc.subcore_barrier()

        base = wid * EPW
        last = base + (FULL - 1) * K

        def issue_idx(i, buf, sem):
            off = pl.multiple_of(
                jnp.minimum(base + i * K, last).astype(jnp.int32), 8)
            pltpu.async_copy(edge_hbm.at[0, pl.ds(off, K)], src_v.at[buf],
                             sem)
            pltpu.async_copy(edge_hbm.at[1, pl.ds(off, K)], dst_v.at[buf],
                             sem)

        def wait_idx(buf, sem):
            pltpu.make_async_copy(edge_hbm.at[0, pl.ds(0, K)],
                                  src_v.at[buf], sem).wait()
            pltpu.make_async_copy(edge_hbm.at[1, pl.ds(0, K)],
                                  dst_v.at[buf], sem).wait()

        issue_idx(0, 0, sia)
        issue_idx(1, 1, sib)
        wait_idx(0, sia)
        pltpu.async_copy(table_hbm.at[src_v.at[0]], rows_a, sga)

        def step(j, _):
            i0 = 2 * j
            wait_idx(1, sib)
            pltpu.async_copy(table_hbm.at[src_v.at[1]], rows_b, sgb)
            pltpu.make_async_copy(table_hbm.at[src_v.at[0]], rows_a,
                                  sga).wait()
            pltpu.sync_copy(rows_a, acc.at[dst_v.at[0]], add=True)
            issue_idx(i0 + 2, 0, sia)
            wait_idx(0, sia)
            pltpu.async_copy(table_hbm.at[src_v.at[0]], rows_a, sga)
            pltpu.make_async_copy(table_hbm.at[src_v.at[1]], rows_b,
                                  sgb).wait()
            pltpu.sync_copy(rows_b, acc.at[dst_v.at[1]], add=True)
            issue_idx(i0 + 3, 1, sib)
            return None
        lax.fori_loop(0, FULL // 2, step, None)

        pltpu.make_async_copy(table_hbm.at[src_v.at[0]], rows_a, sga).wait()
        if FULL % 2:
            pltpu.sync_copy(rows_a, acc.at[dst_v.at[0]], add=True)
        wait_idx(1, sib)
        if TAIL:
            toff = pl.multiple_of(base + FULL * K, 8)
            pltpu.sync_copy(edge_hbm.at[0, pl.ds(toff, TAIL)], st_v)
            pltpu.sync_copy(edge_hbm.at[1, pl.ds(toff, TAIL)], dt_v)
            pltpu.async_copy(table_hbm.at[st_v], rows_t, sga).wait()
            pltpu.sync_copy(rows_t, acc.at[dt_v], add=True)
        plsc.subcore_barrier()

        pltpu.sync_copy(acc.at[pl.ds(s * RPT, RPT)],
                        out_hbm.at[c, pl.ds(s * RPT, RPT)])

    return agg


def _make_deg():
    mesh = plsc.VectorSubcoreMesh(
        core_axis_name="c", subcore_axis_name="s",
        num_cores=NC, num_subcores=NS)

    @functools.partial(
        pl.kernel,
        out_type=jax.ShapeDtypeStruct((NC, NPAD, 16), jnp.float32),
        mesh=mesh,
        compiler_params=pltpu.CompilerParams(use_tc_tiling_on_sc=False),
        scratch_types=[
            pltpu.VMEM((2, K), jnp.int32),
            pltpu.VMEM((TAIL,), jnp.int32),
            pltpu.VMEM((K, 16), jnp.float32),
            pltpu.VMEM((ZROWS, 16), jnp.float32),
            pltpu.VMEM_SHARED((NPAD, 16), jnp.float32),
            pltpu.SemaphoreType.DMA,
            pltpu.SemaphoreType.DMA,
            pltpu.SemaphoreType.DMA,
            pltpu.SemaphoreType.DMA,
        ],
    )
    def deg(edge_hbm, out_hbm, dst_v, dt_v, ones_v, zb_v, acc,
            sia, sib, ssa, ssb):
        c = lax.axis_index("c")
        s = lax.axis_index("s")
        wid = c * NS + s

        ones16 = jnp.ones((16,), jnp.float32)
        zeros16 = jnp.zeros((16,), jnp.float32)

        def fill(j, _):
            ones_v[j, :] = ones16
            return None
        lax.fori_loop(0, K, fill, None)

        def zrow(j, _):
            zb_v[j, :] = zeros16
            return None
        lax.fori_loop(0, ZROWS, zrow, None)

        def zcopy(j, _):
            pltpu.sync_copy(zb_v, acc.at[pl.ds(s * RPT + j * ZROWS, ZROWS)])
            return None
        lax.fori_loop(0, RPT // ZROWS, zcopy, None)
        plsc.subcore_barrier()

        base = wid * EPW
        last = base + (FULL - 1) * K

        def issue_idx(i, buf, sem):
            off = pl.multiple_of(
                jnp.minimum(base + i * K, last).astype(jnp.int32), 8)
            pltpu.async_copy(edge_hbm.at[1, pl.ds(off, K)], dst_v.at[buf],
                             sem)

        def wait_idx(buf, sem):
            pltpu.make_async_copy(edge_hbm.at[1, pl.ds(0, K)],
                                  dst_v.at[buf], sem).wait()

        issue_idx(0, 0, sia)
        issue_idx(1, 1, sib)

        def step(j, _):
            i0 = 2 * j
            wait_idx(0, sia)
            da = pltpu.async_copy(ones_v, acc.at[dst_v.at[0]], ssa,
                                  add=True)
            wait_idx(1, sib)
            db = pltpu.async_copy(ones_v, acc.at[dst_v.at[1]], ssb,
                                  add=True)
            da.wait()
            issue_idx(i0 + 2, 0, sia)
            db.wait()
            issue_idx(i0 + 3, 1, sib)
            return None
        lax.fori_loop(0, FULL // 2, step, None)

        wait_idx(0, sia)
        if FULL % 2:
            pltpu.async_copy(ones_v, acc.at[dst_v.at[0]], ssa,
                             add=True).wait()
        wait_idx(1, sib)
        if TAIL:
            toff = pl.multiple_of(base + FULL * K, 8)
            pltpu.sync_copy(edge_hbm.at[1, pl.ds(toff, TAIL)], dt_v)
            pltpu.async_copy(ones_v.at[pl.ds(0, TAIL)], acc.at[dt_v], ssa,
                             add=True).wait()
        plsc.subcore_barrier()

        pltpu.sync_copy(acc.at[pl.ds(s * RPT, RPT)],
                        out_hbm.at[c, pl.ds(s * RPT, RPT)])

    return deg


def _dinv_of(da_ref, db_ref):
    return lax.rsqrt(da_ref[0][:, :1] + db_ref[0][:, :1] + 1.0)


def _tc1_body(x_ref, g_ref, b_ref, w_ref, p_ref):
    xb = x_ref[...]
    mu = jnp.mean(xb, axis=1, keepdims=True)
    xc = xb - mu
    var = jnp.mean(xc * xc, axis=1, keepdims=True)
    xn = xc * lax.rsqrt(var + EPS) * g_ref[...] + b_ref[...]
    p_ref[...] = jnp.dot(xn, w_ref[...], preferred_element_type=jnp.float32)


def _tc1b_body(p_ref, da_ref, db_ref, hp_ref, s_ref, dv_ref):
    dinv = _dinv_of(da_ref, db_ref)
    hp = p_ref[...] * dinv
    hp_ref[...] = hp
    s_ref[...] = hp * dinv
    dv_ref[...] = dinv * jnp.ones((1, 16), jnp.float32)


def _mid_body(ra_ref, rb_ref, sin_ref, bias_ref, dv_ref, w_ref,
              hp_ref, s_ref):
    dinv = dv_ref[:, :1]
    u = jnp.maximum(
        dinv * (ra_ref[0] + rb_ref[0]) + sin_ref[...] + bias_ref[...],
        0.0)
    hp = jnp.dot(u * dinv, w_ref[...], preferred_element_type=jnp.float32)
    hp_ref[...] = hp
    s_ref[...] = hp * dinv


def _fin_body(ra_ref, rb_ref, sin_ref, bias_ref, dv_ref, out_ref):
    dinv = dv_ref[:, :1]
    out_ref[...] = (dinv * (ra_ref[0] + rb_ref[0])
                    + sin_ref[...] + bias_ref[...])


def _row_spec(d):
    return pl.BlockSpec((B, d), lambda i: (i, 0))


def _full_spec(shape):
    return pl.BlockSpec(shape, lambda i: (0,) * len(shape))


def _tc1(x, g2, b2, W1):
    return pl.pallas_call(
        _tc1_body,
        grid=(N // B,),
        in_specs=[_row_spec(128), _full_spec((1, 128)), _full_spec((1, 128)),
                  _full_spec((128, 128))],
        out_specs=_row_spec(128),
        out_shape=jax.ShapeDtypeStruct((N, 128), jnp.float32),
    )(x, g2, b2, W1)


def _tc1b(p1, deg):
    return pl.pallas_call(
        _tc1b_body,
        grid=(N // B,),
        in_specs=[_row_spec(128), _part_spec(16, 0), _part_spec(16, 1)],
        out_specs=[_row_spec(128), _row_spec(128), _row_spec(16)],
        out_shape=[jax.ShapeDtypeStruct((N, 128), jnp.float32),
                   jax.ShapeDtypeStruct((N, 128), jnp.float32),
                   jax.ShapeDtypeStruct((N, 16), jnp.float32)],
    )(p1, deg, deg)


def _part_spec(d, c):
    if c == 0:
        return pl.BlockSpec((1, B, d), lambda i: (0, i, 0))
    return pl.BlockSpec((1, B, d), lambda i: (1, i, 0))


def _tc_mid(r, sin, bias2, dinv16, W, din, dout):
    return pl.pallas_call(
        _mid_body,
        grid=(N // B,),
        in_specs=[_part_spec(din, 0), _part_spec(din, 1), _row_spec(din),
                  _full_spec((1, din)), _row_spec(16),
                  _full_spec((din, dout))],
        out_specs=[_row_spec(dout), _row_spec(dout)],
        out_shape=[jax.ShapeDtypeStruct((N, dout), jnp.float32),
                   jax.ShapeDtypeStruct((N, dout), jnp.float32)],
    )(r, r, sin, bias2, dinv16, W)


def _tc_fin(r, sin, bias2, dinv16):
    return pl.pallas_call(
        _fin_body,
        grid=(N // B,),
        in_specs=[_part_spec(32, 0), _part_spec(32, 1), _row_spec(32),
                  _full_spec((1, 32)), _row_spec(16)],
        out_specs=pl.BlockSpec((B, 32), lambda i: (i, 0)),
        out_shape=jax.ShapeDtypeStruct((N, 32), jnp.float32),
    )(r, r, sin, bias2, dinv16)


def kernel(x, edge, ln_g, ln_b, W1, b1, W2, b2, W3, b3):
    g2 = ln_g.reshape(1, 128)
    lb2 = ln_b.reshape(1, 128)
    b1_2 = b1.reshape(1, 128)
    b2_2 = b2.reshape(1, 128)
    b3_2 = b3.reshape(1, 32)

    deg = _make_deg()(edge)
    p1 = _tc1(x, g2, lb2, W1)
    h1p, s1, dinv16 = _tc1b(p1, deg)
    r1 = _make_agg(128)(h1p, edge)
    h2p, s2 = _tc_mid(r1, s1, b1_2, dinv16, W2, 128, 128)
    r2 = _make_agg(128)(h2p, edge)
    h3p, s3 = _tc_mid(r2, s2, b2_2, dinv16, W3, 128, 32)
    r3 = _make_agg(32)(h3p, edge)
    return _tc_fin(r3, s3, b3_2, dinv16)


# speedup vs baseline: 1.0480x; 1.0480x over previous
import functools

import jax
import jax.numpy as jnp
from jax import lax
from jax.experimental import pallas as pl
from jax.experimental.pallas import tpu as pltpu
from jax.experimental.pallas import tpu_sc as plsc

N = 10000
NPAD = 10240
E = 320000
NC = 2
NS = 16
NW = NC * NS
EPW = E // NW
K = 128
FULL = EPW // K
TAIL = EPW - FULL * K
RPT = NPAD // NS
ZROWS = 64
B = 2000
EB = 2000
NEB = E // EB
EPS = 1e-5


@functools.lru_cache(maxsize=None)
def _make_agg(D):
    mesh = plsc.VectorSubcoreMesh(
        core_axis_name="c", subcore_axis_name="s",
        num_cores=NC, num_subcores=NS)

    @functools.partial(
        pl.kernel,
        out_type=jax.ShapeDtypeStruct((NC, NPAD, D), jnp.float32),
        mesh=mesh,
        compiler_params=pltpu.CompilerParams(use_tc_tiling_on_sc=False),
        scratch_types=[
            pltpu.VMEM((4, K), jnp.int32),
            pltpu.VMEM((4, K), jnp.int32),
            pltpu.VMEM((K, D), jnp.float32),
            pltpu.VMEM((K, D), jnp.float32),
            pltpu.VMEM((TAIL,), jnp.int32),
            pltpu.VMEM((TAIL,), jnp.int32),
            pltpu.VMEM((TAIL, D), jnp.float32),
            pltpu.VMEM((ZROWS, D), jnp.float32),
            pltpu.VMEM_SHARED((NPAD, D), jnp.float32),
            [pltpu.SemaphoreType.DMA] * 4,
            pltpu.SemaphoreType.DMA,
            pltpu.SemaphoreType.DMA,
        ],
    )
    def agg(table_hbm, edge_hbm, out_hbm,
            src_v, dst_v, rows_a, rows_b, st_v, dt_v, rows_t, zb_v, acc,
            si, sga, sgb):
        c = lax.axis_index("c")
        s = lax.axis_index("s")
        wid = c * NS + s

        zeros16 = jnp.zeros((16,), jnp.float32)

        def zrow(j, _):
            def zlane(k, _):
                zb_v[j, pl.ds(k * 16, 16)] = zeros16
                return None
            return lax.fori_loop(0, D // 16, zlane, None)
        lax.fori_loop(0, ZROWS, zrow, None)

        def zcopy(j, _):
            pltpu.sync_copy(zb_v, acc.at[pl.ds(s * RPT + j * ZROWS, ZROWS)])
            return None
        lax.fori_loop(0, RPT // ZROWS, zcopy, None)
        plsc.subcore_barrier()

        base = wid * EPW
        last = base + (FULL - 1) * K

        def issue_idx(i, buf, sem):
            off = pl.multiple_of(
                jnp.minimum(base + i * K, last).astype(jnp.int32), 8)
            pltpu.async_copy(edge_hbm.at[0, pl.ds(off, K)], src_v.at[buf],
                             sem)
            pltpu.async_copy(edge_hbm.at[1, pl.ds(off, K)], dst_v.at[buf],
                             sem)

        def wait_idx(buf, sem):
            pltpu.make_async_copy(edge_hbm.at[0, pl.ds(0, K)],
                                  src_v.at[buf], sem).wait()
            pltpu.make_async_copy(edge_hbm.at[1, pl.ds(0, K)],
                                  dst_v.at[buf], sem).wait()

        def gather(slot, rows, sem):
            pltpu.async_copy(table_hbm.at[src_v.at[slot]], rows, sem)

        def wait_gather(rows, sem):
            pltpu.make_async_copy(table_hbm.at[src_v.at[0]], rows,
                                  sem).wait()

        def scatter(slot, rows):
            pltpu.sync_copy(rows, acc.at[dst_v.at[slot]], add=True)

        for slot in range(4):
            issue_idx(slot, slot, si[slot])
        wait_idx(0, si[0])
        gather(0, rows_a, sga)

        def step(j, _):
            q0 = 4 * j
            wait_idx(1, si[1])
            gather(1, rows_b, sgb)
            wait_gather(rows_a, sga)
            scatter(0, rows_a)
            issue_idx(q0 + 4, 0, si[0])
            wait_idx(2, si[2])
            gather(2, rows_a, sga)
            wait_gather(rows_b, sgb)
            scatter(1, rows_b)
            issue_idx(q0 + 5, 1, si[1])
            wait_idx(3, si[3])
            gather(3, rows_b, sgb)
            wait_gather(rows_a, sga)
            scatter(2, rows_a)
            issue_idx(q0 + 6, 2, si[2])
            wait_gather(rows_b, sgb)
            scatter(3, rows_b)
            issue_idx(q0 + 7, 3, si[3])
            wait_idx(0, si[0])
            gather(0, rows_a, sga)
            return None
        lax.fori_loop(0, FULL // 4, step, None)

        wait_idx(1, si[1])
        gather(1, rows_b, sgb)
        wait_gather(rows_a, sga)
        scatter(0, rows_a)
        wait_gather(rows_b, sgb)
        scatter(1, rows_b)
        wait_idx(2, si[2])
        wait_idx(3, si[3])
        if TAIL:
            toff = pl.multiple_of(base + FULL * K, 8)
            pltpu.sync_copy(edge_hbm.at[0, pl.ds(toff, TAIL)], st_v)
            pltpu.sync_copy(edge_hbm.at[1, pl.ds(toff, TAIL)], dt_v)
            pltpu.async_copy(table_hbm.at[st_v], rows_t, sga).wait()
            pltpu.sync_copy(rows_t, acc.at[dt_v], add=True)
        plsc.subcore_barrier()

        pltpu.sync_copy(acc.at[pl.ds(s * RPT, RPT)],
                        out_hbm.at[c, pl.ds(s * RPT, RPT)])

    return agg


def _make_deg():
    mesh = plsc.VectorSubcoreMesh(
        core_axis_name="c", subcore_axis_name="s",
        num_cores=NC, num_subcores=NS)

    @functools.partial(
        pl.kernel,
        out_type=jax.ShapeDtypeStruct((NC, NPAD, 16), jnp.float32),
        mesh=mesh,
        compiler_params=pltpu.CompilerParams(use_tc_tiling_on_sc=False),
        scratch_types=[
            pltpu.VMEM((4, K), jnp.int32),
            pltpu.VMEM((TAIL,), jnp.int32),
            pltpu.VMEM((K, 16), jnp.float32),
            pltpu.VMEM((ZROWS, 16), jnp.float32),
            pltpu.VMEM_SHARED((NPAD, 16), jnp.float32),
            [pltpu.SemaphoreType.DMA] * 4,
            [pltpu.SemaphoreType.DMA] * 4,
        ],
    )
    def deg(edge_hbm, out_hbm, dst_v, dt_v, ones_v, zb_v, acc, si, ss):
        c = lax.axis_index("c")
        s = lax.axis_index("s")
        wid = c * NS + s

        ones16 = jnp.ones((16,), jnp.float32)
        zeros16 = jnp.zeros((16,), jnp.float32)

        def fill(j, _):
            ones_v[j, :] = ones16
            return None
        lax.fori_loop(0, K, fill, None)

        def zrow(j, _):
            zb_v[j, :] = zeros16
            return None
        lax.fori_loop(0, ZROWS, zrow, None)

        def zcopy(j, _):
            pltpu.sync_copy(zb_v, acc.at[pl.ds(s * RPT + j * ZROWS, ZROWS)])
            return None
        lax.fori_loop(0, RPT // ZROWS, zcopy, None)
        plsc.subcore_barrier()

        base = wid * EPW
        last = base + (FULL - 1) * K

        def issue_idx(i, buf, sem):
            off = pl.multiple_of(
                jnp.minimum(base + i * K, last).astype(jnp.int32), 8)
            pltpu.async_copy(edge_hbm.at[1, pl.ds(off, K)], dst_v.at[buf],
                             sem)

        def wait_idx(buf, sem):
            pltpu.make_async_copy(edge_hbm.at[1, pl.ds(0, K)],
                                  dst_v.at[buf], sem).wait()

        for slot in range(4):
            issue_idx(slot, slot, si[slot])

        def step(j, _):
            q0 = 4 * j
            for k in range(4):
                wait_idx(k, si[k])
                pltpu.async_copy(ones_v, acc.at[dst_v.at[k]], ss[k],
                                 add=True)
            for k in range(4):
                pltpu.make_async_copy(ones_v, acc.at[dst_v.at[k]],
                                      ss[k]).wait()
                issue_idx(q0 + 4 + k, k, si[k])
            return None
        lax.fori_loop(0, FULL // 4, step, None)

        wait_idx(0, si[0])
        pltpu.async_copy(ones_v, acc.at[dst_v.at[0]], ss[0], add=True)
        wait_idx(1, si[1])
        pltpu.async_copy(ones_v, acc.at[dst_v.at[1]], ss[1], add=True)
        pltpu.make_async_copy(ones_v, acc.at[dst_v.at[0]], ss[0]).wait()
        pltpu.make_async_copy(ones_v, acc.at[dst_v.at[1]], ss[1]).wait()
        wait_idx(2, si[2])
        wait_idx(3, si[3])
        if TAIL:
            toff = pl.multiple_of(base + FULL * K, 8)
            pltpu.sync_copy(edge_hbm.at[1, pl.ds(toff, TAIL)], dt_v)
            pltpu.async_copy(ones_v.at[pl.ds(0, TAIL)], acc.at[dt_v], ss[0],
                             add=True).wait()
        plsc.subcore_barrier()

        pltpu.sync_copy(acc.at[pl.ds(s * RPT, RPT)],
                        out_hbm.at[c, pl.ds(s * RPT, RPT)])

    return deg


def _dinv_of(da_ref, db_ref):
    return lax.rsqrt(da_ref[0][:, :1] + db_ref[0][:, :1] + 1.0)


def _tc1_body(x_ref, g_ref, b_ref, w_ref, p_ref):
    xb = x_ref[...]
    mu = jnp.mean(xb, axis=1, keepdims=True)
    xc = xb - mu
    var = jnp.mean(xc * xc, axis=1, keepdims=True)
    xn = xc * lax.rsqrt(var + EPS) * g_ref[...] + b_ref[...]
    p_ref[...] = jnp.dot(xn, w_ref[...], preferred_element_type=jnp.float32)


def _tc1b_body(p_ref, da_ref, db_ref, hp_ref, s_ref, dv_ref):
    dinv = _dinv_of(da_ref, db_ref)
    hp = p_ref[...] * dinv
    hp_ref[...] = hp
    s_ref[...] = hp * dinv
    dv_ref[...] = dinv * jnp.ones((1, 16), jnp.float32)


def _mid_body(ra_ref, rb_ref, sin_ref, bias_ref, dv_ref, w_ref,
              hp_ref, s_ref):
    dinv = dv_ref[:, :1]
    u = jnp.maximum(
        dinv * (ra_ref[0] + rb_ref[0]) + sin_ref[...] + bias_ref[...],
        0.0)
    hp = jnp.dot(u * dinv, w_ref[...], preferred_element_type=jnp.float32)
    hp_ref[...] = hp
    s_ref[...] = hp * dinv


def _fin_body(ra_ref, rb_ref, sin_ref, bias_ref, dv_ref, out_ref):
    dinv = dv_ref[:, :1]
    out_ref[...] = (dinv * (ra_ref[0] + rb_ref[0])
                    + sin_ref[...] + bias_ref[...])


def _row_spec(d):
    return pl.BlockSpec((B, d), lambda i: (i, 0))


def _full_spec(shape):
    return pl.BlockSpec(shape, lambda i: (0,) * len(shape))


def _tc1(x, g2, b2, W1):
    return pl.pallas_call(
        _tc1_body,
        grid=(N // B,),
        in_specs=[_row_spec(128), _full_spec((1, 128)), _full_spec((1, 128)),
                  _full_spec((128, 128))],
        out_specs=_row_spec(128),
        out_shape=jax.ShapeDtypeStruct((N, 128), jnp.float32),
    )(x, g2, b2, W1)


def _tc1b(p1, deg):
    return pl.pallas_call(
        _tc1b_body,
        grid=(N // B,),
        in_specs=[_row_spec(128), _part_spec(16, 0), _part_spec(16, 1)],
        out_specs=[_row_spec(128), _row_spec(128), _row_spec(16)],
        out_shape=[jax.ShapeDtypeStruct((N, 128), jnp.float32),
                   jax.ShapeDtypeStruct((N, 128), jnp.float32),
                   jax.ShapeDtypeStruct((N, 16), jnp.float32)],
    )(p1, deg, deg)


def _part_spec(d, c):
    if c == 0:
        return pl.BlockSpec((1, B, d), lambda i: (0, i, 0))
    return pl.BlockSpec((1, B, d), lambda i: (1, i, 0))


def _tc_mid(r, sin, bias2, dinv16, W, din, dout):
    return pl.pallas_call(
        _mid_body,
        grid=(N // B,),
        in_specs=[_part_spec(din, 0), _part_spec(din, 1), _row_spec(din),
                  _full_spec((1, din)), _row_spec(16),
                  _full_spec((din, dout))],
        out_specs=[_row_spec(dout), _row_spec(dout)],
        out_shape=[jax.ShapeDtypeStruct((N, dout), jnp.float32),
                   jax.ShapeDtypeStruct((N, dout), jnp.float32)],
    )(r, r, sin, bias2, dinv16, W)


def _tc_fin(r, sin, bias2, dinv16):
    return pl.pallas_call(
        _fin_body,
        grid=(N // B,),
        in_specs=[_part_spec(32, 0), _part_spec(32, 1), _row_spec(32),
                  _full_spec((1, 32)), _row_spec(16)],
        out_specs=pl.BlockSpec((B, 32), lambda i: (i, 0)),
        out_shape=jax.ShapeDtypeStruct((N, 32), jnp.float32),
    )(r, r, sin, bias2, dinv16)


def kernel(x, edge, ln_g, ln_b, W1, b1, W2, b2, W3, b3):
    g2 = ln_g.reshape(1, 128)
    lb2 = ln_b.reshape(1, 128)
    b1_2 = b1.reshape(1, 128)
    b2_2 = b2.reshape(1, 128)
    b3_2 = b3.reshape(1, 32)

    deg = _make_deg()(edge)
    p1 = _tc1(x, g2, lb2, W1)
    h1p, s1, dinv16 = _tc1b(p1, deg)
    r1 = _make_agg(128)(h1p, edge)
    h2p, s2 = _tc_mid(r1, s1, b1_2, dinv16, W2, 128, 128)
    r2 = _make_agg(128)(h2p, edge)
    h3p, s3 = _tc_mid(r2, s2, b2_2, dinv16, W3, 128, 32)
    r3 = _make_agg(32)(h3p, edge)
    return _tc_fin(r3, s3, b3_2, dinv16)


# speedup vs baseline: 1.0753x; 1.0260x over previous
import functools

import jax
import jax.numpy as jnp
from jax import lax
from jax.experimental import pallas as pl
from jax.experimental.pallas import tpu as pltpu
from jax.experimental.pallas import tpu_sc as plsc

N = 10000
NPAD = 10240
E = 320000
NC = 2
NS = 16
NW = NC * NS
EPW = E // NW
K = 128
FULL = EPW // K
TAIL = EPW - FULL * K
RPT = NPAD // NS
ZROWS = 64
B = 2000
EB = 2000
NEB = E // EB
EPS = 1e-5


@functools.lru_cache(maxsize=None)
def _make_agg(D):
    mesh = plsc.VectorSubcoreMesh(
        core_axis_name="c", subcore_axis_name="s",
        num_cores=NC, num_subcores=NS)

    nrows = 4 if D <= 32 else 2

    @functools.partial(
        pl.kernel,
        out_type=jax.ShapeDtypeStruct((NC, NPAD, D), jnp.float32),
        mesh=mesh,
        compiler_params=pltpu.CompilerParams(use_tc_tiling_on_sc=False),
        scratch_types=[
            pltpu.VMEM((4, K), jnp.int32),
            pltpu.VMEM((4, K), jnp.int32),
            [pltpu.VMEM((K, D), jnp.float32)] * nrows,
            pltpu.VMEM((TAIL,), jnp.int32),
            pltpu.VMEM((TAIL,), jnp.int32),
            pltpu.VMEM((TAIL, D), jnp.float32),
            pltpu.VMEM((ZROWS, D), jnp.float32),
            pltpu.VMEM_SHARED((NPAD, D), jnp.float32),
            [pltpu.SemaphoreType.DMA] * 4,
            [pltpu.SemaphoreType.DMA] * nrows,
        ],
    )
    def agg(table_hbm, edge_hbm, out_hbm,
            src_v, dst_v, rows, st_v, dt_v, rows_t, zb_v, acc,
            si, sg):
        rows_a, rows_b = rows[0], rows[1]
        sga, sgb = sg[0], sg[1]
        c = lax.axis_index("c")
        s = lax.axis_index("s")
        wid = c * NS + s

        zeros16 = jnp.zeros((16,), jnp.float32)

        def zrow(j, _):
            def zlane(k, _):
                zb_v[j, pl.ds(k * 16, 16)] = zeros16
                return None
            return lax.fori_loop(0, D // 16, zlane, None)
        lax.fori_loop(0, ZROWS, zrow, None)

        def zcopy(j, _):
            pltpu.sync_copy(zb_v, acc.at[pl.ds(s * RPT + j * ZROWS, ZROWS)])
            return None
        lax.fori_loop(0, RPT // ZROWS, zcopy, None)
        plsc.subcore_barrier()

        base = wid * EPW
        last = base + (FULL - 1) * K

        def issue_idx(i, buf, sem):
            off = pl.multiple_of(
                jnp.minimum(base + i * K, last).astype(jnp.int32), 8)
            pltpu.async_copy(edge_hbm.at[0, pl.ds(off, K)], src_v.at[buf],
                             sem)
            pltpu.async_copy(edge_hbm.at[1, pl.ds(off, K)], dst_v.at[buf],
                             sem)

        def wait_idx(buf, sem):
            pltpu.make_async_copy(edge_hbm.at[0, pl.ds(0, K)],
                                  src_v.at[buf], sem).wait()
            pltpu.make_async_copy(edge_hbm.at[1, pl.ds(0, K)],
                                  dst_v.at[buf], sem).wait()

        def gather(slot, rows, sem):
            pltpu.async_copy(table_hbm.at[src_v.at[slot]], rows, sem)

        def wait_gather(rows, sem):
            pltpu.make_async_copy(table_hbm.at[src_v.at[0]], rows,
                                  sem).wait()

        def scatter(slot, rows):
            pltpu.sync_copy(rows, acc.at[dst_v.at[slot]], add=True)

        for slot in range(4):
            issue_idx(slot, slot, si[slot])

        if nrows == 4:
            def step(j, _):
                q0 = 4 * j
                for k in range(4):
                    wait_idx(k, si[k])
                    gather(k, rows[k], sg[k])
                for k in range(4):
                    wait_gather(rows[k], sg[k])
                    scatter(k, rows[k])
                    issue_idx(q0 + 4 + k, k, si[k])
                return None
            lax.fori_loop(0, FULL // 4, step, None)

            for k in range(2):
                wait_idx(k, si[k])
                gather(k, rows[k], sg[k])
            for k in range(2):
                wait_gather(rows[k], sg[k])
                scatter(k, rows[k])
            wait_idx(2, si[2])
            wait_idx(3, si[3])
        else:
            wait_idx(0, si[0])
            gather(0, rows_a, sga)

            def step(j, _):
                q0 = 4 * j
                wait_idx(1, si[1])
                gather(1, rows_b, sgb)
                wait_gather(rows_a, sga)
                scatter(0, rows_a)
                issue_idx(q0 + 4, 0, si[0])
                wait_idx(2, si[2])
                gather(2, rows_a, sga)
                wait_gather(rows_b, sgb)
                scatter(1, rows_b)
                issue_idx(q0 + 5, 1, si[1])
                wait_idx(3, si[3])
                gather(3, rows_b, sgb)
                wait_gather(rows_a, sga)
                scatter(2, rows_a)
                issue_idx(q0 + 6, 2, si[2])
                wait_gather(rows_b, sgb)
                scatter(3, rows_b)
                issue_idx(q0 + 7, 3, si[3])
                wait_idx(0, si[0])
                gather(0, rows_a, sga)
                return None
            lax.fori_loop(0, FULL // 4, step, None)

            wait_idx(1, si[1])
            gather(1, rows_b, sgb)
            wait_gather(rows_a, sga)
            scatter(0, rows_a)
            wait_gather(rows_b, sgb)
            scatter(1, rows_b)
            wait_idx(2, si[2])
            wait_idx(3, si[3])
        if TAIL:
            toff = pl.multiple_of(base + FULL * K, 8)
            pltpu.sync_copy(edge_hbm.at[0, pl.ds(toff, TAIL)], st_v)
            pltpu.sync_copy(edge_hbm.at[1, pl.ds(toff, TAIL)], dt_v)
            pltpu.async_copy(table_hbm.at[st_v], rows_t, sga).wait()
            pltpu.sync_copy(rows_t, acc.at[dt_v], add=True)
        plsc.subcore_barrier()

        pltpu.sync_copy(acc.at[pl.ds(s * RPT, RPT)],
                        out_hbm.at[c, pl.ds(s * RPT, RPT)])

    return agg


def _make_deg():
    mesh = plsc.VectorSubcoreMesh(
        core_axis_name="c", subcore_axis_name="s",
        num_cores=NC, num_subcores=NS)

    @functools.partial(
        pl.kernel,
        out_type=jax.ShapeDtypeStruct((NC, NPAD, 16), jnp.float32),
        mesh=mesh,
        compiler_params=pltpu.CompilerParams(use_tc_tiling_on_sc=False),
        scratch_types=[
            pltpu.VMEM((4, K), jnp.int32),
            pltpu.VMEM((TAIL,), jnp.int32),
            pltpu.VMEM((K, 16), jnp.float32),
            pltpu.VMEM((ZROWS, 16), jnp.float32),
            pltpu.VMEM_SHARED((NPAD, 16), jnp.float32),
            [pltpu.SemaphoreType.DMA] * 4,
            [pltpu.SemaphoreType.DMA] * 4,
        ],
    )
    def deg(edge_hbm, out_hbm, dst_v, dt_v, ones_v, zb_v, acc, si, ss):
        c = lax.axis_index("c")
        s = lax.axis_index("s")
        wid = c * NS + s

        ones16 = jnp.ones((16,), jnp.float32)
        zeros16 = jnp.zeros((16,), jnp.float32)

        def fill(j, _):
            ones_v[j, :] = ones16
            return None
        lax.fori_loop(0, K, fill, None)

        def zrow(j, _):
            zb_v[j, :] = zeros16
            return None
        lax.fori_loop(0, ZROWS, zrow, None)

        def zcopy(j, _):
            pltpu.sync_copy(zb_v, acc.at[pl.ds(s * RPT + j * ZROWS, ZROWS)])
            return None
        lax.fori_loop(0, RPT // ZROWS, zcopy, None)
        plsc.subcore_barrier()

        base = wid * EPW
        last = base + (FULL - 1) * K

        def issue_idx(i, buf, sem):
            off = pl.multiple_of(
                jnp.minimum(base + i * K, last).astype(jnp.int32), 8)
            pltpu.async_copy(edge_hbm.at[1, pl.ds(off, K)], dst_v.at[buf],
                             sem)

        def wait_idx(buf, sem):
            pltpu.make_async_copy(edge_hbm.at[1, pl.ds(0, K)],
                                  dst_v.at[buf], sem).wait()

        for slot in range(4):
            issue_idx(slot, slot, si[slot])

        def step(j, _):
            q0 = 4 * j
            for k in range(4):
                wait_idx(k, si[k])
                pltpu.async_copy(ones_v, acc.at[dst_v.at[k]], ss[k],
                                 add=True)
            for k in range(4):
                pltpu.make_async_copy(ones_v, acc.at[dst_v.at[k]],
                                      ss[k]).wait()
                issue_idx(q0 + 4 + k, k, si[k])
            return None
        lax.fori_loop(0, FULL // 4, step, None)

        wait_idx(0, si[0])
        pltpu.async_copy(ones_v, acc.at[dst_v.at[0]], ss[0], add=True)
        wait_idx(1, si[1])
        pltpu.async_copy(ones_v, acc.at[dst_v.at[1]], ss[1], add=True)
        pltpu.make_async_copy(ones_v, acc.at[dst_v.at[0]], ss[0]).wait()
        pltpu.make_async_copy(ones_v, acc.at[dst_v.at[1]], ss[1]).wait()
        wait_idx(2, si[2])
        wait_idx(3, si[3])
        if TAIL:
            toff = pl.multiple_of(base + FULL * K, 8)
            pltpu.sync_copy(edge_hbm.at[1, pl.ds(toff, TAIL)], dt_v)
            pltpu.async_copy(ones_v.at[pl.ds(0, TAIL)], acc.at[dt_v], ss[0],
                             add=True).wait()
        plsc.subcore_barrier()

        pltpu.sync_copy(acc.at[pl.ds(s * RPT, RPT)],
                        out_hbm.at[c, pl.ds(s * RPT, RPT)])

    return deg


def _dinv_of(da_ref, db_ref):
    return lax.rsqrt(da_ref[0][:, :1] + db_ref[0][:, :1] + 1.0)


def _tc1_body(x_ref, g_ref, b_ref, w_ref, p_ref):
    xb = x_ref[...]
    mu = jnp.mean(xb, axis=1, keepdims=True)
    xc = xb - mu
    var = jnp.mean(xc * xc, axis=1, keepdims=True)
    xn = xc * lax.rsqrt(var + EPS) * g_ref[...] + b_ref[...]
    p_ref[...] = jnp.dot(xn, w_ref[...], preferred_element_type=jnp.float32)


def _tc1b_body(p_ref, da_ref, db_ref, hp_ref, s_ref, dv_ref):
    dinv = _dinv_of(da_ref, db_ref)
    hp = p_ref[...] * dinv
    hp_ref[...] = hp
    s_ref[...] = hp * dinv
    dv_ref[...] = dinv * jnp.ones((1, 16), jnp.float32)


def _mid_body(ra_ref, rb_ref, sin_ref, bias_ref, dv_ref, w_ref,
              hp_ref, s_ref):
    dinv = dv_ref[:, :1]
    u = jnp.maximum(
        dinv * (ra_ref[0] + rb_ref[0]) + sin_ref[...] + bias_ref[...],
        0.0)
    hp = jnp.dot(u * dinv, w_ref[...], preferred_element_type=jnp.float32)
    hp_ref[...] = hp
    s_ref[...] = hp * dinv


def _fin_body(ra_ref, rb_ref, sin_ref, bias_ref, dv_ref, out_ref):
    dinv = dv_ref[:, :1]
    out_ref[...] = (dinv * (ra_ref[0] + rb_ref[0])
                    + sin_ref[...] + bias_ref[...])


def _row_spec(d):
    return pl.BlockSpec((B, d), lambda i: (i, 0))


def _full_spec(shape):
    return pl.BlockSpec(shape, lambda i: (0,) * len(shape))


def _tc1(x, g2, b2, W1):
    return pl.pallas_call(
        _tc1_body,
        grid=(N // B,),
        in_specs=[_row_spec(128), _full_spec((1, 128)), _full_spec((1, 128)),
                  _full_spec((128, 128))],
        out_specs=_row_spec(128),
        out_shape=jax.ShapeDtypeStruct((N, 128), jnp.float32),
    )(x, g2, b2, W1)


def _tc1b(p1, deg):
    return pl.pallas_call(
        _tc1b_body,
        grid=(N // B,),
        in_specs=[_row_spec(128), _part_spec(16, 0), _part_spec(16, 1)],
        out_specs=[_row_spec(128), _row_spec(128), _row_spec(16)],
        out_shape=[jax.ShapeDtypeStruct((N, 128), jnp.float32),
                   jax.ShapeDtypeStruct((N, 128), jnp.float32),
                   jax.ShapeDtypeStruct((N, 16), jnp.float32)],
    )(p1, deg, deg)


def _part_spec(d, c):
    if c == 0:
        return pl.BlockSpec((1, B, d), lambda i: (0, i, 0))
    return pl.BlockSpec((1, B, d), lambda i: (1, i, 0))


def _tc_mid(r, sin, bias2, dinv16, W, din, dout):
    return pl.pallas_call(
        _mid_body,
        grid=(N // B,),
        in_specs=[_part_spec(din, 0), _part_spec(din, 1), _row_spec(din),
                  _full_spec((1, din)), _row_spec(16),
                  _full_spec((din, dout))],
        out_specs=[_row_spec(dout), _row_spec(dout)],
        out_shape=[jax.ShapeDtypeStruct((N, dout), jnp.float32),
                   jax.ShapeDtypeStruct((N, dout), jnp.float32)],
    )(r, r, sin, bias2, dinv16, W)


def _tc_fin(r, sin, bias2, dinv16):
    return pl.pallas_call(
        _fin_body,
        grid=(N // B,),
        in_specs=[_part_spec(32, 0), _part_spec(32, 1), _row_spec(32),
                  _full_spec((1, 32)), _row_spec(16)],
        out_specs=pl.BlockSpec((B, 32), lambda i: (i, 0)),
        out_shape=jax.ShapeDtypeStruct((N, 32), jnp.float32),
    )(r, r, sin, bias2, dinv16)


def kernel(x, edge, ln_g, ln_b, W1, b1, W2, b2, W3, b3):
    g2 = ln_g.reshape(1, 128)
    lb2 = ln_b.reshape(1, 128)
    b1_2 = b1.reshape(1, 128)
    b2_2 = b2.reshape(1, 128)
    b3_2 = b3.reshape(1, 32)

    deg = _make_deg()(edge)
    p1 = _tc1(x, g2, lb2, W1)
    h1p, s1, dinv16 = _tc1b(p1, deg)
    r1 = _make_agg(128)(h1p, edge)
    h2p, s2 = _tc_mid(r1, s1, b1_2, dinv16, W2, 128, 128)
    r2 = _make_agg(128)(h2p, edge)
    h3p, s3 = _tc_mid(r2, s2, b2_2, dinv16, W3, 128, 32)
    r3 = _make_agg(32)(h3p, edge)
    return _tc_fin(r3, s3, b3_2, dinv16)


# speedup vs baseline: 1.1067x; 1.0292x over previous
import functools

import jax
import jax.numpy as jnp
from jax import lax
from jax.experimental import pallas as pl
from jax.experimental.pallas import tpu as pltpu
from jax.experimental.pallas import tpu_sc as plsc

N = 10000
NPAD = 10240
E = 320000
NC = 2
NS = 16
NW = NC * NS
EPW = E // NW
K = 128
FULL = EPW // K
TAIL = EPW - FULL * K
RPT = NPAD // NS
ZROWS = 64
B = 2000
EB = 2000
NEB = E // EB
EPS = 1e-5


@functools.lru_cache(maxsize=None)
def _make_agg(D):
    mesh = plsc.VectorSubcoreMesh(
        core_axis_name="c", subcore_axis_name="s",
        num_cores=NC, num_subcores=NS)

    nrows = 4 if D <= 32 else 2

    @functools.partial(
        pl.kernel,
        out_type=jax.ShapeDtypeStruct((NC, NPAD, D), jnp.float32),
        mesh=mesh,
        compiler_params=pltpu.CompilerParams(use_tc_tiling_on_sc=False),
        scratch_types=[
            pltpu.VMEM((4, K), jnp.int32),
            pltpu.VMEM((4, K), jnp.int32),
            [pltpu.VMEM((K, D), jnp.float32)] * nrows,
            pltpu.VMEM((TAIL,), jnp.int32),
            pltpu.VMEM((TAIL,), jnp.int32),
            pltpu.VMEM((TAIL, D), jnp.float32),
            pltpu.VMEM((ZROWS, D), jnp.float32),
            pltpu.VMEM_SHARED((NPAD, D), jnp.float32),
            [pltpu.SemaphoreType.DMA] * 4,
            [pltpu.SemaphoreType.DMA] * nrows,
        ],
    )
    def agg(table_hbm, edge_hbm, out_hbm,
            src_v, dst_v, rows, st_v, dt_v, rows_t, zb_v, acc,
            si, sg):
        rows_a, rows_b = rows[0], rows[1]
        sga, sgb = sg[0], sg[1]
        c = lax.axis_index("c")
        s = lax.axis_index("s")
        wid = c * NS + s

        zeros16 = jnp.zeros((16,), jnp.float32)

        def zrow(j, _):
            def zlane(k, _):
                zb_v[j, pl.ds(k * 16, 16)] = zeros16
                return None
            return lax.fori_loop(0, D // 16, zlane, None)
        lax.fori_loop(0, ZROWS, zrow, None)

        def zcopy(j, _):
            pltpu.sync_copy(zb_v, acc.at[pl.ds(s * RPT + j * ZROWS, ZROWS)])
            return None
        lax.fori_loop(0, RPT // ZROWS, zcopy, None)
        plsc.subcore_barrier()

        base = wid * EPW
        last = base + (FULL - 1) * K

        def issue_idx(i, buf, sem):
            off = pl.multiple_of(
                jnp.minimum(base + i * K, last).astype(jnp.int32), 8)
            pltpu.async_copy(edge_hbm.at[0, pl.ds(off, K)], src_v.at[buf],
                             sem)
            pltpu.async_copy(edge_hbm.at[1, pl.ds(off, K)], dst_v.at[buf],
                             sem)

        def wait_idx(buf, sem):
            pltpu.make_async_copy(edge_hbm.at[0, pl.ds(0, K)],
                                  src_v.at[buf], sem).wait()
            pltpu.make_async_copy(edge_hbm.at[1, pl.ds(0, K)],
                                  dst_v.at[buf], sem).wait()

        def gather(slot, rows, sem):
            pltpu.async_copy(table_hbm.at[src_v.at[slot]], rows, sem)

        def wait_gather(rows, sem):
            pltpu.make_async_copy(table_hbm.at[src_v.at[0]], rows,
                                  sem).wait()

        def scatter(slot, rows):
            pltpu.sync_copy(rows, acc.at[dst_v.at[slot]], add=True)

        for slot in range(4 if nrows == 4 else 2):
            issue_idx(slot, slot, si[slot])

        if nrows == 4:
            def step(j, _):
                q0 = 4 * j
                for k in range(4):
                    wait_idx(k, si[k])
                    gather(k, rows[k], sg[k])
                for k in range(4):
                    wait_gather(rows[k], sg[k])
                    scatter(k, rows[k])
                    issue_idx(q0 + 4 + k, k, si[k])
                return None
            lax.fori_loop(0, FULL // 4, step, None)

            for k in range(2):
                wait_idx(k, si[k])
                gather(k, rows[k], sg[k])
            for k in range(2):
                wait_gather(rows[k], sg[k])
                scatter(k, rows[k])
            wait_idx(2, si[2])
            wait_idx(3, si[3])
        else:
            wait_idx(0, si[0])
            gather(0, rows_a, sga)

            def step(j, _):
                i0 = 2 * j
                wait_idx(1, si[1])
                gather(1, rows_b, sgb)
                wait_gather(rows_a, sga)
                scatter(0, rows_a)
                issue_idx(i0 + 2, 0, si[0])
                wait_idx(0, si[0])
                gather(0, rows_a, sga)
                wait_gather(rows_b, sgb)
                scatter(1, rows_b)
                issue_idx(i0 + 3, 1, si[1])
                return None
            lax.fori_loop(0, FULL // 2, step, None)

            wait_gather(rows_a, sga)
            wait_idx(1, si[1])
        if TAIL:
            toff = pl.multiple_of(base + FULL * K, 8)
            pltpu.sync_copy(edge_hbm.at[0, pl.ds(toff, TAIL)], st_v)
            pltpu.sync_copy(edge_hbm.at[1, pl.ds(toff, TAIL)], dt_v)
            pltpu.async_copy(table_hbm.at[st_v], rows_t, sga).wait()
            pltpu.sync_copy(rows_t, acc.at[dt_v], add=True)
        plsc.subcore_barrier()

        pltpu.sync_copy(acc.at[pl.ds(s * RPT, RPT)],
                        out_hbm.at[c, pl.ds(s * RPT, RPT)])

    return agg


def _make_deg():
    mesh = plsc.VectorSubcoreMesh(
        core_axis_name="c", subcore_axis_name="s",
        num_cores=NC, num_subcores=NS)

    @functools.partial(
        pl.kernel,
        out_type=jax.ShapeDtypeStruct((NC, NPAD, 16), jnp.float32),
        mesh=mesh,
        compiler_params=pltpu.CompilerParams(use_tc_tiling_on_sc=False),
        scratch_types=[
            pltpu.VMEM((4, K), jnp.int32),
            pltpu.VMEM((TAIL,), jnp.int32),
            pltpu.VMEM((K, 16), jnp.float32),
            pltpu.VMEM((ZROWS, 16), jnp.float32),
            pltpu.VMEM_SHARED((NPAD, 16), jnp.float32),
            [pltpu.SemaphoreType.DMA] * 4,
            [pltpu.SemaphoreType.DMA] * 4,
        ],
    )
    def deg(edge_hbm, out_hbm, dst_v, dt_v, ones_v, zb_v, acc, si, ss):
        c = lax.axis_index("c")
        s = lax.axis_index("s")
        wid = c * NS + s

        ones16 = jnp.ones((16,), jnp.float32)
        zeros16 = jnp.zeros((16,), jnp.float32)

        def fill(j, _):
            ones_v[j, :] = ones16
            return None
        lax.fori_loop(0, K, fill, None)

        def zrow(j, _):
            zb_v[j, :] = zeros16
            return None
        lax.fori_loop(0, ZROWS, zrow, None)

        def zcopy(j, _):
            pltpu.sync_copy(zb_v, acc.at[pl.ds(s * RPT + j * ZROWS, ZROWS)])
            return None
        lax.fori_loop(0, RPT // ZROWS, zcopy, None)
        plsc.subcore_barrier()

        base = wid * EPW
        last = base + (FULL - 1) * K

        def issue_idx(i, buf, sem):
            off = pl.multiple_of(
                jnp.minimum(base + i * K, last).astype(jnp.int32), 8)
            pltpu.async_copy(edge_hbm.at[1, pl.ds(off, K)], dst_v.at[buf],
                             sem)

        def wait_idx(buf, sem):
            pltpu.make_async_copy(edge_hbm.at[1, pl.ds(0, K)],
                                  dst_v.at[buf], sem).wait()

        for slot in range(4):
            issue_idx(slot, slot, si[slot])

        def step(j, _):
            q0 = 4 * j
            for k in range(4):
                wait_idx(k, si[k])
                pltpu.async_copy(ones_v, acc.at[dst_v.at[k]], ss[k],
                                 add=True)
            for k in range(4):
                pltpu.make_async_copy(ones_v, acc.at[dst_v.at[k]],
                                      ss[k]).wait()
                issue_idx(q0 + 4 + k, k, si[k])
            return None
        lax.fori_loop(0, FULL // 4, step, None)

        wait_idx(0, si[0])
        pltpu.async_copy(ones_v, acc.at[dst_v.at[0]], ss[0], add=True)
        wait_idx(1, si[1])
        pltpu.async_copy(ones_v, acc.at[dst_v.at[1]], ss[1], add=True)
        pltpu.make_async_copy(ones_v, acc.at[dst_v.at[0]], ss[0]).wait()
        pltpu.make_async_copy(ones_v, acc.at[dst_v.at[1]], ss[1]).wait()
        wait_idx(2, si[2])
        wait_idx(3, si[3])
        if TAIL:
            toff = pl.multiple_of(base + FULL * K, 8)
            pltpu.sync_copy(edge_hbm.at[1, pl.ds(toff, TAIL)], dt_v)
            pltpu.async_copy(ones_v.at[pl.ds(0, TAIL)], acc.at[dt_v], ss[0],
                             add=True).wait()
        plsc.subcore_barrier()

        pltpu.sync_copy(acc.at[pl.ds(s * RPT, RPT)],
                        out_hbm.at[c, pl.ds(s * RPT, RPT)])

    return deg


def _dinv_of(da_ref, db_ref):
    return lax.rsqrt(da_ref[0][:, :1] + db_ref[0][:, :1] + 1.0)


def _tc1_body(x_ref, g_ref, b_ref, w_ref, p_ref):
    xb = x_ref[...]
    mu = jnp.mean(xb, axis=1, keepdims=True)
    xc = xb - mu
    var = jnp.mean(xc * xc, axis=1, keepdims=True)
    xn = xc * lax.rsqrt(var + EPS) * g_ref[...] + b_ref[...]
    p_ref[...] = jnp.dot(xn, w_ref[...], preferred_element_type=jnp.float32)


def _tc1b_body(p_ref, da_ref, db_ref, hp_ref, s_ref, dv_ref):
    dinv = _dinv_of(da_ref, db_ref)
    hp = p_ref[...] * dinv
    hp_ref[...] = hp
    s_ref[...] = hp * dinv
    dv_ref[...] = dinv * jnp.ones((1, 16), jnp.float32)


def _mid_body(ra_ref, rb_ref, sin_ref, bias_ref, dv_ref, w_ref,
              hp_ref, s_ref):
    dinv = dv_ref[:, :1]
    u = jnp.maximum(
        dinv * (ra_ref[0] + rb_ref[0]) + sin_ref[...] + bias_ref[...],
        0.0)
    hp = jnp.dot(u * dinv, w_ref[...], preferred_element_type=jnp.float32)
    hp_ref[...] = hp
    s_ref[...] = hp * dinv


def _fin_body(ra_ref, rb_ref, sin_ref, bias_ref, dv_ref, out_ref):
    dinv = dv_ref[:, :1]
    out_ref[...] = (dinv * (ra_ref[0] + rb_ref[0])
                    + sin_ref[...] + bias_ref[...])


def _row_spec(d):
    return pl.BlockSpec((B, d), lambda i: (i, 0))


def _full_spec(shape):
    return pl.BlockSpec(shape, lambda i: (0,) * len(shape))


def _tc1(x, g2, b2, W1):
    return pl.pallas_call(
        _tc1_body,
        grid=(N // B,),
        in_specs=[_row_spec(128), _full_spec((1, 128)), _full_spec((1, 128)),
                  _full_spec((128, 128))],
        out_specs=_row_spec(128),
        out_shape=jax.ShapeDtypeStruct((N, 128), jnp.float32),
    )(x, g2, b2, W1)


def _tc1b(p1, deg):
    return pl.pallas_call(
        _tc1b_body,
        grid=(N // B,),
        in_specs=[_row_spec(128), _part_spec(16, 0), _part_spec(16, 1)],
        out_specs=[_row_spec(128), _row_spec(128), _row_spec(16)],
        out_shape=[jax.ShapeDtypeStruct((N, 128), jnp.float32),
                   jax.ShapeDtypeStruct((N, 128), jnp.float32),
                   jax.ShapeDtypeStruct((N, 16), jnp.float32)],
    )(p1, deg, deg)


def _part_spec(d, c):
    if c == 0:
        return pl.BlockSpec((1, B, d), lambda i: (0, i, 0))
    return pl.BlockSpec((1, B, d), lambda i: (1, i, 0))


def _tc_mid(r, sin, bias2, dinv16, W, din, dout):
    return pl.pallas_call(
        _mid_body,
        grid=(N // B,),
        in_specs=[_part_spec(din, 0), _part_spec(din, 1), _row_spec(din),
                  _full_spec((1, din)), _row_spec(16),
                  _full_spec((din, dout))],
        out_specs=[_row_spec(dout), _row_spec(dout)],
        out_shape=[jax.ShapeDtypeStruct((N, dout), jnp.float32),
                   jax.ShapeDtypeStruct((N, dout), jnp.float32)],
    )(r, r, sin, bias2, dinv16, W)


def _tc_fin(r, sin, bias2, dinv16):
    return pl.pallas_call(
        _fin_body,
        grid=(N // B,),
        in_specs=[_part_spec(32, 0), _part_spec(32, 1), _row_spec(32),
                  _full_spec((1, 32)), _row_spec(16)],
        out_specs=pl.BlockSpec((B, 32), lambda i: (i, 0)),
        out_shape=jax.ShapeDtypeStruct((N, 32), jnp.float32),
    )(r, r, sin, bias2, dinv16)


def kernel(x, edge, ln_g, ln_b, W1, b1, W2, b2, W3, b3):
    g2 = ln_g.reshape(1, 128)
    lb2 = ln_b.reshape(1, 128)
    b1_2 = b1.reshape(1, 128)
    b2_2 = b2.reshape(1, 128)
    b3_2 = b3.reshape(1, 32)

    deg = _make_deg()(edge)
    p1 = _tc1(x, g2, lb2, W1)
    h1p, s1, dinv16 = _tc1b(p1, deg)
    r1 = _make_agg(128)(h1p, edge)
    h2p, s2 = _tc_mid(r1, s1, b1_2, dinv16, W2, 128, 128)
    r2 = _make_agg(128)(h2p, edge)
    h3p, s3 = _tc_mid(r2, s2, b2_2, dinv16, W3, 128, 32)
    r3 = _make_agg(32)(h3p, edge)
    return _tc_fin(r3, s3, b3_2, dinv16)
